# baseline (speedup 1.0000x reference)
import functools

import jax
import jax.numpy as jnp
from jax import lax
from jax.experimental import pallas as pl
from jax.experimental.pallas import tpu as pltpu
from jax.experimental.pallas import tpu_sc as plsc

_BTILE = 8
_KPAD = 8


def _sc_body(logits_hbm, cand_hbm, g_out, f_out, cand_v, idx_v, g_v, f_v, sem,
             *, rows_per, num_k, batch, num_btiles):
    wid = lax.axis_index("s") * 2 + lax.axis_index("c")
    base_row = wid * rows_per
    for kk in range(num_k):
        pltpu.sync_copy(cand_hbm.at[pl.ds(kk * batch + base_row, rows_per)],
                        cand_v.at[pl.ds(kk * rows_per, rows_per)])
    nchunk = rows_per // 16
    for chunk in range(nchunk):
        r = lax.broadcasted_iota(jnp.int32, (16,), 0) + chunk * 16
        cks = [cand_v[pl.ds(kk * rows_per + chunk * 16, 16)]
               for kk in range(num_k)]
        for kk in range(num_k):
            ck = cks[kk]
            fkk = ck >= 0
            for jj in range(kk):
                fkk = jnp.logical_and(fkk, ck != cks[jj])
            safe = jnp.where(ck >= 0, ck, 0)
            o = kk * rows_per + chunk * 16
            idx_v[pl.ds(o, 16)] = (
                ((safe >> 3) * num_btiles + wid) * 1024 + ((safe & 7) << 7) + r)
            f_v[pl.ds(o, 16)] = jnp.where(fkk, 1.0, 0.0)
        for kk in range(num_k, _KPAD):
            o = kk * rows_per + chunk * 16
            f_v[pl.ds(o, 16)] = jnp.zeros((16,), jnp.float32)
            g_v[pl.ds(o, 16)] = jnp.zeros((16,), jnp.float32)
    copies = [pltpu.async_copy(
        logits_hbm.at[idx_v.at[pl.ds(kk * rows_per, rows_per)]],
        g_v.at[pl.ds(kk * rows_per, rows_per)], sem)
        for kk in range(num_k)]
    for cp in copies:
        cp.wait()
    for kk in range(_KPAD):
        pltpu.sync_copy(g_v.at[pl.ds(kk * rows_per, rows_per)],
                        g_out.at[pl.ds(kk * batch + base_row, rows_per)])
        pltpu.sync_copy(f_v.at[pl.ds(kk * rows_per, rows_per)],
                        f_out.at[pl.ds(kk * batch + base_row, rows_per)])


def _sc_gather(logits_flat, cand_flat, batch, num_k):
    rows_per = batch // 32
    mesh = plsc.VectorSubcoreMesh(core_axis_name="c", subcore_axis_name="s")
    body = functools.partial(_sc_body, rows_per=rows_per, num_k=num_k,
                             batch=batch, num_btiles=batch // 128)
    f = pl.kernel(
        body,
        mesh=mesh,
        out_type=[jax.ShapeDtypeStruct((_KPAD * batch,), jnp.float32),
                  jax.ShapeDtypeStruct((_KPAD * batch,), jnp.float32)],
        scratch_types=[
            pltpu.VMEM((num_k * rows_per,), jnp.int32),
            pltpu.VMEM((num_k * rows_per,), jnp.int32),
            pltpu.VMEM((_KPAD * rows_per,), jnp.float32),
            pltpu.VMEM((_KPAD * rows_per,), jnp.float32),
            pltpu.SemaphoreType.DMA,
        ],
    )
    return f(logits_flat, cand_flat)


def _dense_body(x4_ref, g_ref, f_ref, out_ref):
    x = x4_ref[...]
    part = jnp.sum(jnp.log1p(jnp.exp(x)))
    g = g_ref[...]
    f = f_ref[...]
    s = jnp.sum(g * f, axis=0)
    cnt = jnp.maximum(jnp.sum(f, axis=0), 1.0)
    term1 = jnp.log1p(jnp.exp(-(s / cnt)))
    corr = jnp.sum(jnp.where(f > 0, jnp.log1p(jnp.exp(g)), 0.0), axis=0)
    part = part + jnp.sum(term1 - corr)

    @pl.when(pl.program_id(0) == 0)
    def _():
        out_ref[...] = jnp.zeros_like(out_ref)

    out_ref[...] += part.reshape(1, 1)


def _combine_body(g_ref, f_ref, out_ref):
    g = g_ref[...]
    f = f_ref[...]
    s = jnp.sum(g * f, axis=0)
    cnt = jnp.maximum(jnp.sum(f, axis=0), 1.0)
    term1 = jnp.log1p(jnp.exp(-(s / cnt)))
    corr = jnp.sum(jnp.where(f > 0, jnp.log1p(jnp.exp(g)), 0.0), axis=0)
    out_ref[...] = jnp.sum(term1 - corr).reshape(1, 1)


def kernel(logits, candidates):
    b, c = logits.shape
    num_k = candidates.shape[1]
    ctiles, btiles = c // 8, b // 128
    cand_t = candidates.astype(jnp.int32).T.reshape(-1)
    x4 = logits.T.reshape(ctiles, 8, btiles, 128).transpose(0, 2, 1, 3)
    logits_flat = x4.reshape(-1)
    g, f = _sc_gather(logits_flat, cand_t, b, num_k)
    g = g.reshape(_KPAD, b)
    f = f.reshape(_KPAD, b)
    out = pl.pallas_call(
        _dense_body,
        grid=(btiles // _BTILE,),
        in_specs=[pl.BlockSpec((ctiles, _BTILE, 8, 128),
                               lambda i: (0, i, 0, 0)),
                  pl.BlockSpec((_KPAD, _BTILE * 128), lambda i: (0, i)),
                  pl.BlockSpec((_KPAD, _BTILE * 128), lambda i: (0, i))],
        out_specs=pl.BlockSpec((1, 1), lambda i: (0, 0)),
        out_shape=jax.ShapeDtypeStruct((1, 1), jnp.float32),
    )(x4, g, f)
    return out[0, 0] / b


# speedup vs baseline: 1.3413x; 1.3413x over previous
import functools

import jax
import jax.numpy as jnp
from jax import lax
from jax.experimental import pallas as pl
from jax.experimental.pallas import tpu as pltpu
from jax.experimental.pallas import tpu_sc as plsc

_BTILE = 8
_KPAD = 8


def _sc_body(logits_hbm, cand_hbm, g_out, f_out, cand_v, idx_v, g_v, f_v, sem,
             *, rows_per, num_k, batch, num_btiles):
    wid = lax.axis_index("s") * 2 + lax.axis_index("c")
    base_row = wid * rows_per
    for kk in range(num_k):
        pltpu.sync_copy(cand_hbm.at[pl.ds(kk * batch + base_row, rows_per)],
                        cand_v.at[pl.ds(kk * rows_per, rows_per)])
    nchunk = rows_per // 16
    for chunk in range(nchunk):
        r = lax.broadcasted_iota(jnp.int32, (16,), 0) + chunk * 16
        cks = [cand_v[pl.ds(kk * rows_per + chunk * 16, 16)]
               for kk in range(num_k)]
        for kk in range(num_k):
            ck = cks[kk]
            fkk = ck >= 0
            for jj in range(kk):
                fkk = jnp.logical_and(fkk, ck != cks[jj])
            safe = jnp.where(ck >= 0, ck, 0)
            o = kk * rows_per + chunk * 16
            idx_v[pl.ds(o, 16)] = (
                ((safe >> 3) * num_btiles + wid) * 1024 + ((safe & 7) << 7) + r)
            f_v[pl.ds(o, 16)] = jnp.where(fkk, 1.0, 0.0)
        for kk in range(num_k, _KPAD):
            o = kk * rows_per + chunk * 16
            f_v[pl.ds(o, 16)] = jnp.zeros((16,), jnp.float32)
            g_v[pl.ds(o, 16)] = jnp.zeros((16,), jnp.float32)
    copies = [pltpu.async_copy(
        logits_hbm.at[idx_v.at[pl.ds(kk * rows_per, rows_per)]],
        g_v.at[pl.ds(kk * rows_per, rows_per)], sem)
        for kk in range(num_k)]
    for cp in copies:
        cp.wait()
    for kk in range(_KPAD):
        pltpu.sync_copy(g_v.at[pl.ds(kk * rows_per, rows_per)],
                        g_out.at[pl.ds(kk * batch + base_row, rows_per)])
        pltpu.sync_copy(f_v.at[pl.ds(kk * rows_per, rows_per)],
                        f_out.at[pl.ds(kk * batch + base_row, rows_per)])


def _sc_gather(logits_flat, cand_flat, batch, num_k):
    rows_per = batch // 32
    mesh = plsc.VectorSubcoreMesh(core_axis_name="c", subcore_axis_name="s")
    body = functools.partial(_sc_body, rows_per=rows_per, num_k=num_k,
                             batch=batch, num_btiles=batch // 128)
    f = pl.kernel(
        body,
        mesh=mesh,
        out_type=[jax.ShapeDtypeStruct((_KPAD * batch,), jnp.float32),
                  jax.ShapeDtypeStruct((_KPAD * batch,), jnp.float32)],
        scratch_types=[
            pltpu.VMEM((num_k * rows_per,), jnp.int32),
            pltpu.VMEM((num_k * rows_per,), jnp.int32),
            pltpu.VMEM((_KPAD * rows_per,), jnp.float32),
            pltpu.VMEM((_KPAD * rows_per,), jnp.float32),
            pltpu.SemaphoreType.DMA,
        ],
    )
    return f(logits_flat, cand_flat)


def _dense_body(x4_ref, out_ref):
    x = x4_ref[...]
    part = jnp.sum(jnp.log1p(jnp.exp(x)))

    @pl.when(pl.program_id(0) == 0)
    def _():
        out_ref[...] = jnp.zeros_like(out_ref)

    out_ref[...] += part.reshape(1, 1)


def _combine_body(g_ref, f_ref, dense_ref, out_ref, *, kpad, inv_batch):
    rows = g_ref.shape[0]
    g = g_ref[...].reshape(kpad, rows // kpad, 128)
    f = f_ref[...].reshape(kpad, rows // kpad, 128)
    s = jnp.sum(g * f, axis=0)
    cnt = jnp.maximum(jnp.sum(f, axis=0), 1.0)
    term1 = jnp.log1p(jnp.exp(-(s / cnt)))
    corr = jnp.sum(jnp.where(f > 0, jnp.log1p(jnp.exp(g)), 0.0), axis=0)
    part = jnp.sum(term1 - corr).reshape(1, 1)
    out_ref[...] = (dense_ref[...] + part) * inv_batch


def kernel(logits, candidates):
    b, c = logits.shape
    num_k = candidates.shape[1]
    ctiles, btiles = c // 8, b // 128
    cand_t = candidates.astype(jnp.int32).T.reshape(-1)
    x4 = logits.T.reshape(ctiles, 8, btiles, 128).transpose(0, 2, 1, 3)
    logits_flat = x4.reshape(-1)
    g, f = _sc_gather(logits_flat, cand_t, b, num_k)
    g = g.reshape(_KPAD * b // 128, 128)
    f = f.reshape(_KPAD * b // 128, 128)
    dense = pl.pallas_call(
        _dense_body,
        grid=(btiles // _BTILE,),
        in_specs=[pl.BlockSpec((ctiles, _BTILE, 8, 128),
                               lambda i: (0, i, 0, 0))],
        out_specs=pl.BlockSpec((1, 1), lambda i: (0, 0)),
        out_shape=jax.ShapeDtypeStruct((1, 1), jnp.float32),
    )(x4)
    gr = _KPAD * b // 128
    out = pl.pallas_call(
        functools.partial(_combine_body, kpad=_KPAD, inv_batch=1.0 / b),
        in_specs=[pl.BlockSpec((gr, 128), lambda: (0, 0)),
                  pl.BlockSpec((gr, 128), lambda: (0, 0)),
                  pl.BlockSpec((1, 1), lambda: (0, 0))],
        out_specs=pl.BlockSpec((1, 1), lambda: (0, 0)),
        out_shape=jax.ShapeDtypeStruct((1, 1), jnp.float32),
    )(g, f, dense)
    return out[0, 0]
